# baseline (speedup 1.0000x reference)
import functools
import math

import jax
import jax.numpy as jnp
from jax import lax
from jax.experimental import pallas as pl
from jax.experimental.pallas import tpu as pltpu
from jax.experimental.pallas import tpu_sc as plsc

_D = 64
_NC, _NS = 2, 16
_NW = _NC * _NS
_LANES = 16
_SCALE = math.sqrt(_D)


@functools.partial(jax.jit, static_argnames=("B", "C"))
def _embed_lookup(tokens_flat, table, *, B, C):
    b_per_w = B // _NW
    nchunks = b_per_w // C
    mesh = plsc.VectorSubcoreMesh(
        core_axis_name="c", subcore_axis_name="s",
        num_cores=_NC, num_subcores=_NS)

    @functools.partial(
        pl.kernel,
        out_type=jax.ShapeDtypeStruct((B, _D), jnp.float32),
        mesh=mesh,
        compiler_params=pltpu.CompilerParams(use_tc_tiling_on_sc=False),
        scratch_types=[
            pltpu.VMEM((C,), jnp.int32),
            pltpu.VMEM((C, _D), jnp.float32),
            pltpu.SemaphoreType.DMA,
        ],
    )
    def k(tokens_hbm, table_hbm, out_hbm, idx_v, rows_v, sem):
        wid = lax.axis_index("s") * _NC + lax.axis_index("c")
        base = wid * b_per_w

        @pl.loop(0, nchunks)
        def _chunk(g):
            off = base + g * C
            pltpu.sync_copy(tokens_hbm.at[pl.ds(off, C)], idx_v)
            pltpu.async_copy(table_hbm.at[idx_v], rows_v, sem).wait()

            @pl.loop(0, C)
            def _row(r):
                for j in range(_D // _LANES):
                    sl = pl.ds(j * _LANES, _LANES)
                    rows_v[r, sl] = rows_v[r, sl] * _SCALE

            pltpu.sync_copy(rows_v, out_hbm.at[pl.ds(off, C)])

    return k(tokens_flat, table)


def kernel(tokens, table):
    B = tokens.shape[0] * tokens.shape[1]
    flat = tokens.reshape(B).astype(jnp.int32)
    out = _embed_lookup(flat, table, B=B, C=512)
    return out.reshape(tokens.shape[0], tokens.shape[1], _D)


# speedup vs baseline: 1.1934x; 1.1934x over previous
import functools
import math

import jax
import jax.numpy as jnp
from jax import lax
from jax.experimental import pallas as pl
from jax.experimental.pallas import tpu as pltpu
from jax.experimental.pallas import tpu_sc as plsc

_D = 64
_NC, _NS = 2, 16
_NW = _NC * _NS
_LANES = 16
_SCALE = math.sqrt(_D)


@functools.partial(jax.jit, static_argnames=("B", "C"))
def _embed_lookup(tokens_flat, table, *, B, C):
    b_per_w = B // _NW
    nchunks = b_per_w // C
    assert nchunks % 2 == 0
    mesh = plsc.VectorSubcoreMesh(
        core_axis_name="c", subcore_axis_name="s",
        num_cores=_NC, num_subcores=_NS)

    @functools.partial(
        pl.kernel,
        out_type=jax.ShapeDtypeStruct((B, _D), jnp.float32),
        mesh=mesh,
        compiler_params=pltpu.CompilerParams(use_tc_tiling_on_sc=False),
        scratch_types=[
            pltpu.VMEM((2, C), jnp.int32),
            pltpu.VMEM((2, C, _D), jnp.float32),
            pltpu.SemaphoreType.DMA,
            pltpu.SemaphoreType.DMA,
            pltpu.SemaphoreType.DMA,
            pltpu.SemaphoreType.DMA,
        ],
    )
    def k(tokens_hbm, table_hbm, out_hbm, idx_v, rows_v, gs0, gs1, os0, os1):
        gsems = (gs0, gs1)
        osems = (os0, os1)
        wid = lax.axis_index("s") * _NC + lax.axis_index("c")
        base = wid * b_per_w

        def start_gather(buf, g, gsem):
            off = base + g * C
            pltpu.sync_copy(tokens_hbm.at[pl.ds(off, C)], idx_v.at[buf])
            pltpu.async_copy(table_hbm.at[idx_v.at[buf]], rows_v.at[buf], gsem)

        start_gather(0, 0, gs0)

        @pl.loop(0, nchunks, step=2)
        def _outer(G):
            for b in range(2):
                g = G + b
                nb = 1 - b

                @pl.when(g + 1 < nchunks)
                def _start_next():
                    @pl.when(g >= 1)
                    def _drain():
                        pltpu.make_async_copy(
                            rows_v.at[nb], out_hbm.at[pl.ds(base, C)],
                            osems[nb]).wait()
                    start_gather(nb, g + 1, gsems[nb])

                pltpu.make_async_copy(
                    table_hbm.at[idx_v.at[b]], rows_v.at[b], gsems[b]).wait()

                @pl.loop(0, C, unroll=8)
                def _scale(r):
                    for j in range(_D // _LANES):
                        sl = pl.ds(j * _LANES, _LANES)
                        rows_v[b, r, sl] = rows_v[b, r, sl] * _SCALE

                pltpu.async_copy(
                    rows_v.at[b], out_hbm.at[pl.ds(base + g * C, C)], osems[b])

        pltpu.make_async_copy(
            rows_v.at[0], out_hbm.at[pl.ds(base, C)], os0).wait()
        pltpu.make_async_copy(
            rows_v.at[1], out_hbm.at[pl.ds(base, C)], os1).wait()

    return k(tokens_flat, table)


def kernel(tokens, table):
    B = tokens.shape[0] * tokens.shape[1]
    flat = tokens.reshape(B).astype(jnp.int32)
    out = _embed_lookup(flat, table, B=B, C=512)
    return out.reshape(tokens.shape[0], tokens.shape[1], _D)
